# Initial kernel scaffold; baseline (speedup 1.0000x reference)
import jax
import jax.numpy as jnp
from jax.experimental import pallas as pl


def kernel(x, adj, W1_0, b1_0, W2_0, b2_0, W1_1, b1_1, W2_1, b2_1, lin_W, lin_b):
    raise NotImplementedError("write your pallas kernel here")


# speedup vs baseline: 598.4151x; 598.4151x over previous
import functools

import jax
import jax.numpy as jnp
from jax.experimental import pallas as pl
from jax.experimental.pallas import tpu as pltpu

N = 10000
D = 128
K_BLK = 400
NK = N // K_BLK


def _gin_kernel(adj_ref, h_ref, w1_ref, b1_ref, w2_ref, b2_ref,
                lwa_ref, lwb_ref, lb_ref, out_ref, acc_ref, *, last_layer):
    k = pl.program_id(0)

    @pl.when(k == 0)
    def _init():
        acc_ref[...] = jnp.zeros_like(acc_ref)

    a = adj_ref[...].astype(jnp.bfloat16)
    hb = h_ref[pl.ds(k * K_BLK, K_BLK), :].astype(jnp.bfloat16)
    acc_ref[...] += jax.lax.dot_general(
        a, hb, (((0,), (0,)), ((), ())),
        preferred_element_type=jnp.float32)

    @pl.when(k == NK - 1)
    def _epilogue():
        hin = h_ref[...]
        z = hin + acc_ref[...]
        z = jnp.maximum(z @ w1_ref[...] + b1_ref[...], 0.0)
        z = z @ w2_ref[...] + b2_ref[...]
        hl = jnp.maximum(z, 0.0)
        if last_layer:
            f = hin @ lwa_ref[...] + hl @ lwb_ref[...] + lb_ref[...]
            out_ref[...] = jnp.maximum(f, 0.0)
        else:
            out_ref[...] = hl


def _gin_layer(adj, h, W1, b1, W2, b2, lwa, lwb, lb, last_layer):
    wspec = pl.BlockSpec((D, D), lambda k: (0, 0))
    bspec = pl.BlockSpec((1, D), lambda k: (0, 0))
    fullspec = pl.BlockSpec((N, D), lambda k: (0, 0))
    return pl.pallas_call(
        functools.partial(_gin_kernel, last_layer=last_layer),
        grid=(NK,),
        in_specs=[
            pl.BlockSpec((K_BLK, N), lambda k: (k, 0)),
            fullspec,
            wspec, bspec, wspec, bspec, wspec, wspec, bspec,
        ],
        out_specs=fullspec,
        out_shape=jax.ShapeDtypeStruct((N, D), jnp.float32),
        scratch_shapes=[pltpu.VMEM((N, D), jnp.float32)],
        compiler_params=pltpu.CompilerParams(
            dimension_semantics=("arbitrary",)),
    )(adj, h, W1, b1.reshape(1, D), W2, b2.reshape(1, D), lwa, lwb, lb)


@jax.jit
def kernel(x, adj, W1_0, b1_0, W2_0, b2_0, W1_1, b1_1, W2_1, b2_1, lin_W, lin_b):
    lwa = lin_W[:D]
    lwb = lin_W[D:]
    lb = lin_b.reshape(1, D)
    h1 = _gin_layer(adj, x, W1_0, b1_0, W2_0, b2_0, lwa, lwb, lb, False)
    out = _gin_layer(adj, h1, W1_1, b1_1, W2_1, b2_1, lwa, lwb, lb, True)
    return out
